# baseline (speedup 1.0000x reference)
import jax
import jax.numpy as jnp
from jax.experimental import pallas as pl
from jax.experimental.pallas import tpu as pltpu

_N = 4096
_K = 64
_KP = 128
_BS = 128
_MC = 1024


def _chamfer_body(x_ref, y_ref, out_ref, acc_ref, b2_ref, cacc_ref):
    b_b = pl.program_id(0)
    b_i = pl.program_id(1)
    nb = pl.num_programs(0)
    ni = pl.num_programs(1)

    @pl.when((b_b == 0) & (b_i == 0))
    def _init_acc():
        acc_ref[0, 0] = 0.0

    @pl.when(b_i == 0)
    def _build_b2():
        y = y_ref[0]
        yn = jnp.sum(y * y, axis=1, keepdims=True)
        b2_ref[...] = jnp.concatenate(
            [y, jnp.ones((_N, 1), jnp.float32), yn,
             jnp.zeros((_N, _KP - _K - 2), jnp.float32)],
            axis=1).astype(jnp.bfloat16)
        cacc_ref[...] = jnp.full((8, _N), jnp.inf, jnp.float32)

    x = x_ref[0]
    xn = jnp.sum(x * x, axis=1, keepdims=True)
    a2 = jnp.concatenate(
        [x * -2.0, xn, jnp.ones((_MC, 1), jnp.float32),
         jnp.zeros((_MC, _KP - _K - 2), jnp.float32)],
        axis=1).astype(jnp.bfloat16)

    racc = jnp.full((_MC, _BS), jnp.inf, jnp.float32)
    for s in range(_N // _BS):
        b2s = b2_ref[pl.ds(s * _BS, _BS), :]
        d = jax.lax.dot_general(
            a2, b2s, (((1,), (1,)), ((), ())),
            preferred_element_type=jnp.float32)
        racc = jnp.minimum(racc, d)
        cp8 = jnp.min(d.reshape(_MC // 8, 8, _BS), axis=0)
        csl = (slice(None), pl.ds(s * _BS, _BS))
        cacc_ref[csl] = jnp.minimum(cacc_ref[csl], cp8)

    acc_ref[0, 0] += jnp.sum(jnp.min(racc, axis=1))

    @pl.when(b_i == ni - 1)
    def _fin_cols():
        acc_ref[0, 0] += jnp.sum(jnp.min(cacc_ref[...], axis=0))

    @pl.when((b_b == nb - 1) & (b_i == ni - 1))
    def _write_out():
        out_ref[...] = jnp.full((1, 1), acc_ref[0, 0], jnp.float32)


def kernel(input, output):
    nb, n, k = input.shape
    res = pl.pallas_call(
        _chamfer_body,
        grid=(nb, n // _MC),
        in_specs=[
            pl.BlockSpec((1, _MC, k), lambda b, i: (b, i, 0)),
            pl.BlockSpec((1, n, k), lambda b, i: (b, 0, 0)),
        ],
        out_specs=pl.BlockSpec((1, 1), lambda b, i: (0, 0)),
        out_shape=jax.ShapeDtypeStruct((1, 1), jnp.float32),
        scratch_shapes=[
            pltpu.SMEM((1, 1), jnp.float32),
            pltpu.VMEM((_N, _KP), jnp.bfloat16),
            pltpu.VMEM((8, _N), jnp.float32),
        ],
    )(input, output)
    return res[0, 0]


# speedup vs baseline: 1.1195x; 1.1195x over previous
import jax
import jax.numpy as jnp
from jax.experimental import pallas as pl
from jax.experimental.pallas import tpu as pltpu

_N = 4096
_K = 64
_KP = 128
_BS = 128
_MC = 1024


def _chamfer_body(x_ref, y_ref, out_ref, acc_ref, a2_ref, b2_ref, cacc_ref):
    b_b = pl.program_id(0)
    nb = pl.num_programs(0)

    @pl.when(b_b == 0)
    def _init_acc():
        acc_ref[0, 0] = 0.0

    x = x_ref[0]
    y = y_ref[0]
    xn = jnp.sum(x * x, axis=1, keepdims=True)
    yn = jnp.sum(y * y, axis=1, keepdims=True)
    ones = jnp.ones((_N, 1), jnp.float32)
    zeros = jnp.zeros((_N, _KP - _K - 2), jnp.float32)
    a2_ref[...] = jnp.concatenate(
        [x * -2.0, xn, ones, zeros], axis=1).astype(jnp.bfloat16)
    b2_ref[...] = jnp.concatenate(
        [y, ones, yn, zeros], axis=1).astype(jnp.bfloat16)

    loss = jnp.float32(0.0)
    for m in range(_N // _MC):
        a2m = a2_ref[pl.ds(m * _MC, _MC), :]
        racc = jnp.full((_MC, _BS), jnp.inf, jnp.float32)
        for s in range(_N // _BS):
            b2s = b2_ref[pl.ds(s * _BS, _BS), :]
            d = jax.lax.dot_general(
                a2m, b2s, (((1,), (1,)), ((), ())),
                preferred_element_type=jnp.float32)
            racc = jnp.minimum(racc, d)
            cp8 = jnp.min(d.reshape(_MC // 8, 8, _BS), axis=0)
            csl = (slice(None), pl.ds(s * _BS, _BS))
            if m == 0:
                cacc_ref[csl] = cp8
            else:
                cacc_ref[csl] = jnp.minimum(cacc_ref[csl], cp8)
        loss += jnp.sum(jnp.min(racc, axis=1))

    acc_ref[0, 0] += loss + jnp.sum(jnp.min(cacc_ref[...], axis=0))

    @pl.when(b_b == nb - 1)
    def _write_out():
        out_ref[...] = jnp.full((1, 1), acc_ref[0, 0], jnp.float32)


def kernel(input, output):
    nb, n, k = input.shape
    res = pl.pallas_call(
        _chamfer_body,
        grid=(nb,),
        in_specs=[
            pl.BlockSpec((1, n, k), lambda b: (b, 0, 0)),
            pl.BlockSpec((1, n, k), lambda b: (b, 0, 0)),
        ],
        out_specs=pl.BlockSpec((1, 1), lambda b: (0, 0)),
        out_shape=jax.ShapeDtypeStruct((1, 1), jnp.float32),
        scratch_shapes=[
            pltpu.SMEM((1, 1), jnp.float32),
            pltpu.VMEM((_N, _KP), jnp.bfloat16),
            pltpu.VMEM((_N, _KP), jnp.bfloat16),
            pltpu.VMEM((8, _N), jnp.float32),
        ],
    )(input, output)
    return res[0, 0]
